# baseline (speedup 1.0000x reference)
import functools

import jax
import jax.numpy as jnp
from jax import lax
from jax.experimental import pallas as pl
from jax.experimental.pallas import tpu as pltpu
from jax.experimental.pallas import tpu_sc as plsc

VOCAB = 100000
HIDDEN = 128
MAX_POS = 512
SEQ = 200
BATCH = 1024
EPS = 1e-12

_NC = 2
_NS = 16
_NW = _NC * _NS
_L = 16

_TOKENS = BATCH * SEQ
_BSIZES = (640, 128, 128, 128)
_K = len(_BSIZES)
_BOFFS = tuple(sum(_BSIZES[:i]) for i in range(_K))
_CHUNK = 80
_NBUF = 2
_NVR = _CHUNK // _L


def _sc_gather_type(table, types, ids_flat, tt_flat, koff):
  mesh = plsc.VectorSubcoreMesh(core_axis_name="c", subcore_axis_name="s")
  ctok = _BSIZES[koff] * SEQ
  toff = _BOFFS[koff] * SEQ
  tpw = ctok // _NW
  nchunk = tpw // _CHUNK

  @functools.partial(
      pl.kernel,
      out_type=jax.ShapeDtypeStruct((ctok, HIDDEN), jnp.float32),
      mesh=mesh,
      scratch_types=[
          pltpu.VMEM((tpw,), jnp.int32),
          pltpu.VMEM((tpw,), jnp.int32),
          pltpu.VMEM((2, HIDDEN), jnp.float32),
          pltpu.VMEM((_NBUF, _CHUNK, HIDDEN), jnp.float32),
          pltpu.SemaphoreType.DMA,
          pltpu.SemaphoreType.DMA,
      ],
      name=f"sc_gather_{koff}",
  )
  def k(table_hbm, types_hbm, idx_hbm, tt_hbm, out_hbm, idx_v, tt_v, ty_v,
        rows_v, gsem, wsem):
    wid = lax.axis_index("s") * _NC + lax.axis_index("c")
    base = wid * tpw
    pltpu.sync_copy(idx_hbm.at[pl.ds(toff + base, tpw)], idx_v)
    pltpu.sync_copy(tt_hbm.at[pl.ds(toff + base, tpw)], tt_v)
    pltpu.sync_copy(types_hbm, ty_v)
    t0 = [ty_v[0, pl.ds(h * _L, _L)] for h in range(HIDDEN // _L)]
    dt = [ty_v[1, pl.ds(h * _L, _L)] - t0[h] for h in range(HIDDEN // _L)]

    def start_gw(j, b):
      idx_c = idx_v.at[pl.ds(j * _CHUNK, _CHUNK)]
      return pltpu.async_copy(table_hbm.at[idx_c], rows_v.at[b], gsem)

    def wait_gw(b):
      idx_c = idx_v.at[pl.ds(0, _CHUNK)]
      pltpu.make_async_copy(table_hbm.at[idx_c], rows_v.at[b], gsem).wait()

    def start_wb(j, b):
      dst = out_hbm.at[pl.ds(base + j * _CHUNK, _CHUNK)]
      return pltpu.async_copy(rows_v.at[b], dst, wsem)

    def wait_wb(b):
      dst = out_hbm.at[pl.ds(base, _CHUNK)]
      pltpu.make_async_copy(rows_v.at[b], dst, wsem).wait()

    def add_type(j, b):
      for v in range(_NVR):
        ttv = tt_v[pl.ds(j * _CHUNK + v * _L, _L)].astype(jnp.float32)
        for lane in range(_L):
          idxv = jnp.full((_L,), lane, jnp.int32)
          bc = jnp.take_along_axis(
              ttv, idxv, axis=0,
              mode=lax.GatherScatterMode.PROMISE_IN_BOUNDS)
          r = v * _L + lane
          for h in range(HIDDEN // _L):
            sl = pl.ds(h * _L, _L)
            rows_v[b, r, sl] = rows_v[b, r, sl] + (t0[h] + bc * dt[h])

    start_gw(0, 0)

    def body(j2, _):
      for sub in range(_NBUF):
        j = j2 * _NBUF + sub
        b = sub
        bn = (sub + 1) % _NBUF
        wait_gw(b)
        if sub == _NBUF - 1:
          wait_wb(bn)

          @pl.when(j2 < nchunk // _NBUF - 1)
          def _():
            start_gw(j + 1, bn)
        else:
          @pl.when(j2 > 0)
          def _():
            wait_wb(bn)

          start_gw(j + 1, bn)
        add_type(j, b)
        start_wb(j, b)
      return 0

    lax.fori_loop(0, nchunk // _NBUF, body, 0)
    for b in range(1, _NBUF):
      wait_wb(b)

  return k(table, types, ids_flat, tt_flat)


_BB = 64


def _ln_body(words_ref, pos_ref, gamma_ref, beta_ref, out_ref):
  emb = words_ref[...] + pos_ref[...][None]
  mean = jnp.mean(emb, axis=-1, keepdims=True)
  var = jnp.mean(jnp.square(emb - mean), axis=-1, keepdims=True)
  normed = (emb - mean) * lax.rsqrt(var + EPS)
  out_ref[...] = normed * gamma_ref[0, :] + beta_ref[0, :]


def _ln_first_kernel(words_ref, pos_ref, gamma_ref, beta_ref, out_ref):
  _ln_body(words_ref, pos_ref, gamma_ref, beta_ref, out_ref)


def _ln_next_kernel(buf_ref, words_ref, pos_ref, gamma_ref, beta_ref,
                    out_ref):
  del buf_ref
  _ln_body(words_ref, pos_ref, gamma_ref, beta_ref, out_ref)


def _tc_layernorm_chunk(koff, buf, words, W_pos_seq, gamma2, beta2):
  nblk = _BSIZES[koff] // _BB
  blk0 = _BOFFS[koff] // _BB
  base_specs = [
      pl.BlockSpec((_BB, SEQ, HIDDEN), lambda i: (i, 0, 0)),
      pl.BlockSpec((SEQ, HIDDEN), lambda i: (0, 0)),
      pl.BlockSpec((1, HIDDEN), lambda i: (0, 0)),
      pl.BlockSpec((1, HIDDEN), lambda i: (0, 0)),
  ]
  out_spec = pl.BlockSpec((_BB, SEQ, HIDDEN),
                          lambda i, b0=blk0: (b0 + i, 0, 0))
  out_shape = jax.ShapeDtypeStruct((BATCH, SEQ, HIDDEN), jnp.float32)
  if buf is None:
    return pl.pallas_call(
        _ln_first_kernel,
        grid=(nblk,),
        in_specs=base_specs,
        out_specs=out_spec,
        out_shape=out_shape,
    )(words, W_pos_seq, gamma2, beta2)
  return pl.pallas_call(
      _ln_next_kernel,
      grid=(nblk,),
      in_specs=[pl.BlockSpec(memory_space=pl.ANY)] + base_specs,
      out_specs=out_spec,
      out_shape=out_shape,
      input_output_aliases={0: 0},
  )(buf, words, W_pos_seq, gamma2, beta2)


def kernel(input_ids, token_type_ids, W_word, W_pos, W_type, gamma, beta):
  ids_flat = input_ids.reshape(-1).astype(jnp.int32)
  tt_flat = token_type_ids.reshape(-1).astype(jnp.int32)
  W_pos_seq = W_pos[:SEQ]
  gamma2 = gamma.reshape(1, HIDDEN)
  beta2 = beta.reshape(1, HIDDEN)
  words = [
      _sc_gather_type(W_word, W_type, ids_flat, tt_flat,
                      k).reshape(_BSIZES[k], SEQ, HIDDEN)
      for k in range(_K)
  ]
  buf = None
  for k in range(_K):
    buf = _tc_layernorm_chunk(k, buf, words[k], W_pos_seq, gamma2, beta2)
  return buf


# speedup vs baseline: 1.0379x; 1.0379x over previous
import functools

import jax
import jax.numpy as jnp
from jax import lax
from jax.experimental import pallas as pl
from jax.experimental.pallas import tpu as pltpu
from jax.experimental.pallas import tpu_sc as plsc

VOCAB = 100000
HIDDEN = 128
MAX_POS = 512
SEQ = 200
BATCH = 1024
EPS = 1e-12

_NC = 2
_NS = 16
_NW = _NC * _NS
_L = 16

_TOKENS = BATCH * SEQ
_BSIZES = (640, 256, 128)
_K = len(_BSIZES)
_BOFFS = tuple(sum(_BSIZES[:i]) for i in range(_K))
_CHUNK = 80
_NBUF = 2
_NVR = _CHUNK // _L


def _sc_gather_type(table, types, ids_flat, tt_flat, koff):
  mesh = plsc.VectorSubcoreMesh(core_axis_name="c", subcore_axis_name="s")
  ctok = _BSIZES[koff] * SEQ
  toff = _BOFFS[koff] * SEQ
  tpw = ctok // _NW
  nchunk = tpw // _CHUNK

  @functools.partial(
      pl.kernel,
      out_type=jax.ShapeDtypeStruct((ctok, HIDDEN), jnp.float32),
      mesh=mesh,
      scratch_types=[
          pltpu.VMEM((tpw,), jnp.int32),
          pltpu.VMEM((tpw,), jnp.int32),
          pltpu.VMEM((2, HIDDEN), jnp.float32),
          pltpu.VMEM((_NBUF, _CHUNK, HIDDEN), jnp.float32),
          pltpu.SemaphoreType.DMA,
          pltpu.SemaphoreType.DMA,
      ],
      name=f"sc_gather_{koff}",
  )
  def k(table_hbm, types_hbm, idx_hbm, tt_hbm, out_hbm, idx_v, tt_v, ty_v,
        rows_v, gsem, wsem):
    wid = lax.axis_index("s") * _NC + lax.axis_index("c")
    base = wid * tpw
    pltpu.sync_copy(idx_hbm.at[pl.ds(toff + base, tpw)], idx_v)
    pltpu.sync_copy(tt_hbm.at[pl.ds(toff + base, tpw)], tt_v)
    pltpu.sync_copy(types_hbm, ty_v)
    t0 = [ty_v[0, pl.ds(h * _L, _L)] for h in range(HIDDEN // _L)]
    dt = [ty_v[1, pl.ds(h * _L, _L)] - t0[h] for h in range(HIDDEN // _L)]

    def start_gw(j, b):
      idx_c = idx_v.at[pl.ds(j * _CHUNK, _CHUNK)]
      return pltpu.async_copy(table_hbm.at[idx_c], rows_v.at[b], gsem)

    def wait_gw(b):
      idx_c = idx_v.at[pl.ds(0, _CHUNK)]
      pltpu.make_async_copy(table_hbm.at[idx_c], rows_v.at[b], gsem).wait()

    def start_wb(j, b):
      dst = out_hbm.at[pl.ds(base + j * _CHUNK, _CHUNK)]
      return pltpu.async_copy(rows_v.at[b], dst, wsem)

    def wait_wb(b):
      dst = out_hbm.at[pl.ds(base, _CHUNK)]
      pltpu.make_async_copy(rows_v.at[b], dst, wsem).wait()

    def add_type(j, b):
      for v in range(_NVR):
        ttv = tt_v[pl.ds(j * _CHUNK + v * _L, _L)].astype(jnp.float32)
        for lane in range(_L):
          idxv = jnp.full((_L,), lane, jnp.int32)
          bc = jnp.take_along_axis(
              ttv, idxv, axis=0,
              mode=lax.GatherScatterMode.PROMISE_IN_BOUNDS)
          r = v * _L + lane
          for h in range(HIDDEN // _L):
            sl = pl.ds(h * _L, _L)
            rows_v[b, r, sl] = rows_v[b, r, sl] + (t0[h] + bc * dt[h])

    start_gw(0, 0)

    def body(j2, _):
      for sub in range(_NBUF):
        j = j2 * _NBUF + sub
        b = sub
        bn = (sub + 1) % _NBUF
        wait_gw(b)
        if sub == _NBUF - 1:
          wait_wb(bn)

          @pl.when(j2 < nchunk // _NBUF - 1)
          def _():
            start_gw(j + 1, bn)
        else:
          @pl.when(j2 > 0)
          def _():
            wait_wb(bn)

          start_gw(j + 1, bn)
        add_type(j, b)
        start_wb(j, b)
      return 0

    lax.fori_loop(0, nchunk // _NBUF, body, 0)
    for b in range(1, _NBUF):
      wait_wb(b)

  return k(table, types, ids_flat, tt_flat)


_BB = 64


def _ln_body(words_ref, pos_ref, gamma_ref, beta_ref, out_ref):
  emb = words_ref[...] + pos_ref[...][None]
  mean = jnp.mean(emb, axis=-1, keepdims=True)
  var = jnp.mean(jnp.square(emb - mean), axis=-1, keepdims=True)
  normed = (emb - mean) * lax.rsqrt(var + EPS)
  out_ref[...] = normed * gamma_ref[0, :] + beta_ref[0, :]


def _ln_first_kernel(words_ref, pos_ref, gamma_ref, beta_ref, out_ref):
  _ln_body(words_ref, pos_ref, gamma_ref, beta_ref, out_ref)


def _ln_next_kernel(buf_ref, words_ref, pos_ref, gamma_ref, beta_ref,
                    out_ref):
  del buf_ref
  _ln_body(words_ref, pos_ref, gamma_ref, beta_ref, out_ref)


def _tc_layernorm_chunk(koff, buf, words, W_pos_seq, gamma2, beta2):
  nblk = _BSIZES[koff] // _BB
  blk0 = _BOFFS[koff] // _BB
  base_specs = [
      pl.BlockSpec((_BB, SEQ, HIDDEN), lambda i: (i, 0, 0)),
      pl.BlockSpec((SEQ, HIDDEN), lambda i: (0, 0)),
      pl.BlockSpec((1, HIDDEN), lambda i: (0, 0)),
      pl.BlockSpec((1, HIDDEN), lambda i: (0, 0)),
  ]
  out_spec = pl.BlockSpec((_BB, SEQ, HIDDEN),
                          lambda i, b0=blk0: (b0 + i, 0, 0))
  out_shape = jax.ShapeDtypeStruct((BATCH, SEQ, HIDDEN), jnp.float32)
  if buf is None:
    return pl.pallas_call(
        _ln_first_kernel,
        grid=(nblk,),
        in_specs=base_specs,
        out_specs=out_spec,
        out_shape=out_shape,
    )(words, W_pos_seq, gamma2, beta2)
  return pl.pallas_call(
      _ln_next_kernel,
      grid=(nblk,),
      in_specs=[pl.BlockSpec(memory_space=pl.ANY)] + base_specs,
      out_specs=out_spec,
      out_shape=out_shape,
      input_output_aliases={0: 0},
  )(buf, words, W_pos_seq, gamma2, beta2)


def kernel(input_ids, token_type_ids, W_word, W_pos, W_type, gamma, beta):
  ids_flat = input_ids.reshape(-1).astype(jnp.int32)
  tt_flat = token_type_ids.reshape(-1).astype(jnp.int32)
  W_pos_seq = W_pos[:SEQ]
  gamma2 = gamma.reshape(1, HIDDEN)
  beta2 = beta.reshape(1, HIDDEN)
  words = [
      _sc_gather_type(W_word, W_type, ids_flat, tt_flat,
                      k).reshape(_BSIZES[k], SEQ, HIDDEN)
      for k in range(_K)
  ]
  buf = None
  for k in range(_K):
    buf = _tc_layernorm_chunk(k, buf, words[k], W_pos_seq, gamma2, beta2)
  return buf
